# baseline (speedup 1.0000x reference)
import jax
import jax.numpy as jnp
import numpy as np
from jax.experimental import pallas as pl
from jax.experimental.pallas import tpu as pltpu
from jax.experimental.pallas import tpu_sc as plsc

_ROWS = 128
_ROW_LEN = 160000
_BS = _ROWS // 2
_NUM_WORKERS = 32
_CHUNK = 4000
_CPR = _ROW_LEN // _CHUNK
_FLAT = _ROWS * _CPR
_IPW = _FLAT // _NUM_WORKERS
_G = 8
_GROUPS = _IPW // _G
_NB = 2

_PERM = (22, 18, 6, 26, 21, 45, 60, 39, 61, 49, 38, 27, 32, 57, 10, 63,
         35, 20, 24, 56, 52, 40, 51, 42, 55, 4, 31, 14, 0, 43, 34, 3,
         50, 5, 17, 37, 28, 2, 41, 23, 58, 44, 54, 48, 46, 36, 1, 8,
         16, 33, 30, 7, 19, 15, 9, 62, 13, 11, 59, 47, 25, 53, 12, 29)
_SRC_ROWS = _PERM + tuple(range(_BS, _ROWS))
_SRC_FLAT = np.asarray(
    [_SRC_ROWS[f // _CPR] * _CPR + f % _CPR for f in range(_FLAT)],
    dtype=np.int32,
)


def _sc_permute_copy(srcf, idxf):
    mesh = plsc.VectorSubcoreMesh(core_axis_name="c", subcore_axis_name="s")

    @pl.kernel(
        out_type=jax.ShapeDtypeStruct((_FLAT, _CHUNK), jnp.float32),
        mesh=mesh,
        compiler_params=pltpu.CompilerParams(use_tc_tiling_on_sc=False),
        scratch_types=[
            pltpu.VMEM((_NB, _G, _CHUNK), jnp.float32),
            pltpu.VMEM((_IPW,), jnp.int32),
        ]
        + [pltpu.SemaphoreType.DMA] * (2 * _NB),
    )
    def k(src_hbm, idx_hbm, out_hbm, bufs, idx_v, *sems):
        in_sems = sems[:_NB]
        out_sems = sems[_NB:]
        wid = jax.lax.axis_index("s") * 2 + jax.lax.axis_index("c")
        base = wid * _IPW

        pltpu.sync_copy(idx_hbm.at[pl.ds(base, _IPW)], idx_v)

        def in_copy(b, g):
            return pltpu.make_async_copy(
                src_hbm.at[idx_v.at[pl.ds(g * _G, _G)]],
                bufs.at[b],
                in_sems[b],
            )

        def out_copy(b, g):
            return pltpu.make_async_copy(
                bufs.at[b],
                out_hbm.at[pl.ds(base + g * _G, _G)],
                out_sems[b],
            )

        for b in range(_NB):
            in_copy(b, jnp.int32(b)).start()

        @pl.loop(0, _GROUPS, step=_NB)
        def _(t):
            for b in range(_NB):
                g = t + b
                in_copy(b, g).wait()
                out_copy(b, g).start()
            for b in range(_NB):
                g = t + b
                nxt = g + _NB

                @pl.when(nxt < _GROUPS)
                def _(b=b, g=g, nxt=nxt):
                    out_copy(b, g).wait()
                    in_copy(b, nxt).start()

        for b in range(_NB):
            out_copy(b, jnp.int32(_GROUPS - _NB + b)).wait()

    return k(srcf, idxf)


def kernel(sources):
    srcf = sources.reshape(_FLAT, _CHUNK)
    idxf = jnp.asarray(_SRC_FLAT)
    out = _sc_permute_copy(srcf, idxf)
    return out.reshape(2, _BS, 1, _ROW_LEN)


# speedup vs baseline: 1.0044x; 1.0044x over previous
import jax
import jax.numpy as jnp
import numpy as np
from jax.experimental import pallas as pl
from jax.experimental.pallas import tpu as pltpu
from jax.experimental.pallas import tpu_sc as plsc

_ROWS = 128
_ROW_LEN = 160000
_BS = _ROWS // 2
_NUM_WORKERS = 32
_CHUNK = 2000
_CPR = _ROW_LEN // _CHUNK
_FLAT = _ROWS * _CPR
_IPW = _FLAT // _NUM_WORKERS
_G = 8
_GROUPS = _IPW // _G
_NB = 4

_PERM = (22, 18, 6, 26, 21, 45, 60, 39, 61, 49, 38, 27, 32, 57, 10, 63,
         35, 20, 24, 56, 52, 40, 51, 42, 55, 4, 31, 14, 0, 43, 34, 3,
         50, 5, 17, 37, 28, 2, 41, 23, 58, 44, 54, 48, 46, 36, 1, 8,
         16, 33, 30, 7, 19, 15, 9, 62, 13, 11, 59, 47, 25, 53, 12, 29)
_SRC_ROWS = _PERM + tuple(range(_BS, _ROWS))
_SRC_FLAT = np.asarray(
    [_SRC_ROWS[f // _CPR] * _CPR + f % _CPR for f in range(_FLAT)],
    dtype=np.int32,
)


def _sc_permute_copy(srcf, idxf):
    mesh = plsc.VectorSubcoreMesh(core_axis_name="c", subcore_axis_name="s")

    @pl.kernel(
        out_type=jax.ShapeDtypeStruct((_FLAT, _CHUNK), jnp.float32),
        mesh=mesh,
        compiler_params=pltpu.CompilerParams(use_tc_tiling_on_sc=False),
        scratch_types=[
            pltpu.VMEM((_NB, _G, _CHUNK), jnp.float32),
            pltpu.VMEM((_IPW,), jnp.int32),
        ]
        + [pltpu.SemaphoreType.DMA] * (2 * _NB),
    )
    def k(src_hbm, idx_hbm, out_hbm, bufs, idx_v, *sems):
        in_sems = sems[:_NB]
        out_sems = sems[_NB:]
        wid = jax.lax.axis_index("s") * 2 + jax.lax.axis_index("c")
        base = wid * _IPW

        pltpu.sync_copy(idx_hbm.at[pl.ds(base, _IPW)], idx_v)

        def in_copy(b, g):
            return pltpu.make_async_copy(
                src_hbm.at[idx_v.at[pl.ds(g * _G, _G)]],
                bufs.at[b],
                in_sems[b],
            )

        def out_copy(b, g):
            return pltpu.make_async_copy(
                bufs.at[b],
                out_hbm.at[pl.ds(base + g * _G, _G)],
                out_sems[b],
            )

        for b in range(_NB):
            in_copy(b, jnp.int32(b)).start()

        @pl.loop(0, _GROUPS, step=_NB)
        def _(t):
            for b in range(_NB):
                g = t + b
                in_copy(b, g).wait()
                out_copy(b, g).start()
            for b in range(_NB):
                g = t + b
                nxt = g + _NB

                @pl.when(nxt < _GROUPS)
                def _(b=b, g=g, nxt=nxt):
                    out_copy(b, g).wait()
                    in_copy(b, nxt).start()

        for b in range(_NB):
            out_copy(b, jnp.int32(_GROUPS - _NB + b)).wait()

    return k(srcf, idxf)


def kernel(sources):
    srcf = sources.reshape(_FLAT, _CHUNK)
    idxf = jnp.asarray(_SRC_FLAT)
    out = _sc_permute_copy(srcf, idxf)
    return out.reshape(2, _BS, 1, _ROW_LEN)


# speedup vs baseline: 1.0390x; 1.0345x over previous
import jax
import jax.numpy as jnp
from jax.experimental import pallas as pl
from jax.experimental.pallas import tpu as pltpu
from jax.experimental.pallas import tpu_sc as plsc

_ROWS = 128
_ROW_LEN = 160000
_BS = _ROWS // 2
_NUM_WORKERS = 32
_RPW = _ROWS // _NUM_WORKERS
_NBUF = 8
_CHUNK = 8000
_CPR = _ROW_LEN // _CHUNK
_ITEMS = _RPW * _CPR

_PERM = (22, 18, 6, 26, 21, 45, 60, 39, 61, 49, 38, 27, 32, 57, 10, 63,
         35, 20, 24, 56, 52, 40, 51, 42, 55, 4, 31, 14, 0, 43, 34, 3,
         50, 5, 17, 37, 28, 2, 41, 23, 58, 44, 54, 48, 46, 36, 1, 8,
         16, 33, 30, 7, 19, 15, 9, 62, 13, 11, 59, 47, 25, 53, 12, 29)
_SRC_ROWS = _PERM + tuple(range(_BS, _ROWS))


def _sc_permute_copy(src2d):
    mesh = plsc.VectorSubcoreMesh(core_axis_name="c", subcore_axis_name="s")

    @pl.kernel(
        out_type=jax.ShapeDtypeStruct((_ROWS, _ROW_LEN), jnp.float32),
        mesh=mesh,
        compiler_params=pltpu.CompilerParams(use_tc_tiling_on_sc=False),
        scratch_types=[
            pltpu.VMEM((_NBUF, _CHUNK), jnp.float32),
            pltpu.SMEM((_RPW,), jnp.int32),
        ]
        + [pltpu.SemaphoreType.DMA] * (2 * _NBUF),
    )
    def k(src_hbm, out_hbm, bufs, srows, *sems):
        in_sems = sems[:_NBUF]
        out_sems = sems[_NBUF:]
        wid = jax.lax.axis_index("s") * 2 + jax.lax.axis_index("c")
        dst_base = wid * _RPW

        for w in range(_NUM_WORKERS):

            @pl.when(wid == w)
            def _(w=w):
                for i in range(_RPW):
                    srows[i] = _SRC_ROWS[w * _RPW + i]

        def in_copy(b, it):
            r = jax.lax.div(it, _CPR)
            off = jax.lax.mul(jax.lax.rem(it, _CPR), _CHUNK)
            return pltpu.make_async_copy(
                src_hbm.at[srows[r], pl.ds(off, _CHUNK)],
                bufs.at[b],
                in_sems[b],
            )

        def out_copy(b, it):
            r = jax.lax.div(it, _CPR)
            off = jax.lax.mul(jax.lax.rem(it, _CPR), _CHUNK)
            return pltpu.make_async_copy(
                bufs.at[b],
                out_hbm.at[dst_base + r, pl.ds(off, _CHUNK)],
                out_sems[b],
            )

        for b in range(_NBUF):
            in_copy(b, jnp.int32(b)).start()

        @pl.loop(0, _ITEMS, step=_NBUF)
        def _(t):
            for b in range(_NBUF):
                it = t + b
                in_copy(b, it).wait()
                out_copy(b, it).start()
            for b in range(_NBUF):
                it = t + b
                nxt = it + _NBUF

                @pl.when(nxt < _ITEMS)
                def _(b=b, it=it, nxt=nxt):
                    out_copy(b, it).wait()
                    in_copy(b, nxt).start()

        for b in range(_NBUF):
            out_copy(b, jnp.int32(_ITEMS - _NBUF + b)).wait()

    return k(src2d)


def kernel(sources):
    src2d = sources.reshape(_ROWS, _ROW_LEN)
    out = _sc_permute_copy(src2d)
    return out.reshape(2, _BS, 1, _ROW_LEN)
